# Initial kernel scaffold; baseline (speedup 1.0000x reference)
import jax
import jax.numpy as jnp
from jax.experimental import pallas as pl


def kernel(backtrack_potentials, city_to_insert_probs, edge_to_insert_probs, current_tour):
    raise NotImplementedError("write your pallas kernel here")


# speedup vs baseline: 1.0494x; 1.0494x over previous
import numpy as np
import jax
import jax.numpy as jnp
from jax import lax
from jax.experimental import pallas as pl

B = 4096
N = 1000
BB = 128


def _np_threefry2x32(k1, k2, x0, x1):
    k1, k2 = np.uint32(k1), np.uint32(k2)
    ks = [k1, k2, np.uint32(k1 ^ k2 ^ np.uint32(0x1BD11BDA))]
    rots = [[13, 15, 26, 6], [17, 29, 16, 24]]
    x0 = (x0 + ks[0]).astype(np.uint32)
    x1 = (x1 + ks[1]).astype(np.uint32)
    for i in range(5):
        for r in rots[i % 2]:
            x0 = (x0 + x1).astype(np.uint32)
            x1 = ((x1 << np.uint32(r)) | (x1 >> np.uint32(32 - r))).astype(np.uint32)
            x1 = (x0 ^ x1).astype(np.uint32)
        x0 = (x0 + ks[(i + 1) % 3]).astype(np.uint32)
        x1 = (x1 + ks[(i + 2) % 3] + np.uint32(i + 1)).astype(np.uint32)
    return x0, x1


def _subkeys_of_42():
    idx = np.arange(3, dtype=np.uint64)
    hi = (idx >> np.uint64(32)).astype(np.uint32)
    lo = (idx & np.uint64(0xFFFFFFFF)).astype(np.uint32)
    o0, o1 = _np_threefry2x32(np.uint32(0), np.uint32(42), hi, lo)
    return [(int(o0[i]), int(o1[i])) for i in range(3)]


_KB, _KC, _KE = _subkeys_of_42()

_TINY = np.float32(np.finfo(np.float32).tiny)
_ONE_MINUS_TINY = np.float32(np.float32(1.0) - _TINY)
_LOG_1E9 = np.float32(np.log(np.float32(1e-9)))
_ROTS = (13, 15, 26, 6, 17, 29, 16, 24, 13, 15, 26, 6, 17, 29, 16, 24, 13, 15, 26, 6)


def _as_i32(x):
    return np.array(x, dtype=np.uint32).view(np.int32)[()]


def _rotl(x, r):
    return lax.shift_left(x, np.int32(r)) | lax.shift_right_logical(x, np.int32(32 - r))


def _gumbel_bits(key, idx):
    k1, k2 = np.uint32(key[0]), np.uint32(key[1])
    k3 = np.uint32(k1 ^ k2 ^ np.uint32(0x1BD11BDA))
    kseq = [_as_i32(k1), _as_i32(k2), _as_i32(k3)]
    x0 = jnp.full(idx.shape, kseq[0], dtype=jnp.int32)
    x1 = idx + kseq[1]
    for i in range(5):
        for r in _ROTS[i * 4 : i * 4 + 4]:
            x0 = x0 + x1
            x1 = _rotl(x1, r)
            x1 = x0 ^ x1
        x0 = x0 + kseq[(i + 1) % 3]
        x1 = x1 + kseq[(i + 2) % 3] + np.int32(i + 1)
    bits = x0 ^ x1
    fb = lax.shift_right_logical(bits, np.int32(9)) | np.int32(0x3F800000)
    fl = lax.bitcast_convert_type(fb, jnp.float32) - np.float32(1.0)
    u = fl * _ONE_MINUS_TINY + _TINY
    u = jnp.maximum(_TINY, u)
    return -jnp.log(-jnp.log(u))


def _first_argmax(s, col):
    m = jnp.max(s, axis=1, keepdims=True)
    return jnp.min(jnp.where(s == m, col, np.int32(N)), axis=1)


def _body(pot_ref, pc_ref, pe_ref, tour_ref, bt_ref, city_ref, es_ref, ee_ref):
    i = pl.program_id(0)
    row = lax.broadcasted_iota(jnp.int32, (BB, N), 0)
    col = lax.broadcasted_iota(jnp.int32, (BB, N), 1)
    idx = (i * np.int32(BB) + row) * np.int32(N) + col

    g_b = _gumbel_bits(_KB, idx)
    bt_ref[...] = _first_argmax(pot_ref[...] + g_b, col)

    g_c = _gumbel_bits(_KC, idx)
    city = _first_argmax(jnp.log(pc_ref[...]) + g_c, col)
    city_ref[...] = city

    g_e = _gumbel_bits(_KE, idx)
    s_e = jnp.where(col == city[:, None], _LOG_1E9, jnp.log(pe_ref[...])) + g_e
    ie = _first_argmax(s_e, col)
    es_ref[...] = ie

    tour = tour_ref[...]
    pos = jnp.min(jnp.where(tour == ie[:, None], col, np.int32(N)), axis=1)
    nxt = jnp.where(pos == np.int32(N - 1), np.int32(0), pos + np.int32(1))
    ee_ref[...] = jnp.sum(jnp.where(col == nxt[:, None], tour, np.int32(0)), axis=1)


def kernel(backtrack_potentials, city_to_insert_probs, edge_to_insert_probs, current_tour):
    in_spec = pl.BlockSpec((BB, N), lambda i: (i, 0))
    out_spec = pl.BlockSpec((BB,), lambda i: (i,))
    out_shape = jax.ShapeDtypeStruct((B,), jnp.int32)
    bt, city, es, ee = pl.pallas_call(
        _body,
        grid=(B // BB,),
        in_specs=[in_spec] * 4,
        out_specs=[out_spec] * 4,
        out_shape=[out_shape] * 4,
    )(backtrack_potentials, city_to_insert_probs, edge_to_insert_probs, current_tour)
    return bt, city, jnp.stack([es, ee], axis=1)


# speedup vs baseline: 1.0689x; 1.0185x over previous
import numpy as np
import jax
import jax.numpy as jnp
from jax import lax
from jax.experimental import pallas as pl

B = 4096
N = 1000
BB = 256


def _np_threefry2x32(k1, k2, x0, x1):
    k1, k2 = np.uint32(k1), np.uint32(k2)
    ks = [k1, k2, np.uint32(k1 ^ k2 ^ np.uint32(0x1BD11BDA))]
    rots = [[13, 15, 26, 6], [17, 29, 16, 24]]
    x0 = (x0 + ks[0]).astype(np.uint32)
    x1 = (x1 + ks[1]).astype(np.uint32)
    for i in range(5):
        for r in rots[i % 2]:
            x0 = (x0 + x1).astype(np.uint32)
            x1 = ((x1 << np.uint32(r)) | (x1 >> np.uint32(32 - r))).astype(np.uint32)
            x1 = (x0 ^ x1).astype(np.uint32)
        x0 = (x0 + ks[(i + 1) % 3]).astype(np.uint32)
        x1 = (x1 + ks[(i + 2) % 3] + np.uint32(i + 1)).astype(np.uint32)
    return x0, x1


def _subkeys_of_42():
    idx = np.arange(3, dtype=np.uint64)
    hi = (idx >> np.uint64(32)).astype(np.uint32)
    lo = (idx & np.uint64(0xFFFFFFFF)).astype(np.uint32)
    o0, o1 = _np_threefry2x32(np.uint32(0), np.uint32(42), hi, lo)
    return [(int(o0[i]), int(o1[i])) for i in range(3)]


_KB, _KC, _KE = _subkeys_of_42()

_TINY = np.float32(np.finfo(np.float32).tiny)
_ONE_MINUS_TINY = np.float32(np.float32(1.0) - _TINY)
_LOG_1E9 = np.float32(np.log(np.float32(1e-9)))
_ROTS = (13, 15, 26, 6, 17, 29, 16, 24, 13, 15, 26, 6, 17, 29, 16, 24, 13, 15, 26, 6)


def _as_i32(x):
    return np.array(x, dtype=np.uint32).view(np.int32)[()]


def _rotl(x, r):
    return lax.shift_left(x, np.int32(r)) | lax.shift_right_logical(x, np.int32(32 - r))


def _gumbel_bits(key, idx):
    k1, k2 = np.uint32(key[0]), np.uint32(key[1])
    k3 = np.uint32(k1 ^ k2 ^ np.uint32(0x1BD11BDA))
    kseq = [_as_i32(k1), _as_i32(k2), _as_i32(k3)]
    x0 = jnp.full(idx.shape, kseq[0], dtype=jnp.int32)
    x1 = idx + kseq[1]
    for i in range(5):
        for r in _ROTS[i * 4 : i * 4 + 4]:
            x0 = x0 + x1
            x1 = _rotl(x1, r)
            x1 = x0 ^ x1
        x0 = x0 + kseq[(i + 1) % 3]
        x1 = x1 + kseq[(i + 2) % 3] + np.int32(i + 1)
    bits = x0 ^ x1
    fb = lax.shift_right_logical(bits, np.int32(9)) | np.int32(0x3F800000)
    fl = lax.bitcast_convert_type(fb, jnp.float32) - np.float32(1.0)
    u = fl + _TINY
    return -jnp.log(-jnp.log(u))


def _first_argmax(s, col):
    m = jnp.max(s, axis=1, keepdims=True)
    return jnp.min(jnp.where(s == m, col, np.int32(N)), axis=1)


def _body(pot_ref, pc_ref, pe_ref, tour_ref, bt_ref, city_ref, es_ref, ee_ref):
    i = pl.program_id(0)
    row = lax.broadcasted_iota(jnp.int32, (BB, N), 0)
    col = lax.broadcasted_iota(jnp.int32, (BB, N), 1)
    idx = (i * np.int32(BB) + row) * np.int32(N) + col

    g_b = _gumbel_bits(_KB, idx)
    bt_ref[...] = _first_argmax(pot_ref[...] + g_b, col)

    g_c = _gumbel_bits(_KC, idx)
    city = _first_argmax(jnp.log(pc_ref[...]) + g_c, col)
    city_ref[...] = city

    g_e = _gumbel_bits(_KE, idx)
    s_e = jnp.where(col == city[:, None], _LOG_1E9, jnp.log(pe_ref[...])) + g_e
    ie = _first_argmax(s_e, col)
    es_ref[...] = ie

    tour = tour_ref[...]
    pos = jnp.min(jnp.where(tour == ie[:, None], col, np.int32(N)), axis=1)
    nxt = jnp.where(pos == np.int32(N - 1), np.int32(0), pos + np.int32(1))
    ee_ref[...] = jnp.sum(jnp.where(col == nxt[:, None], tour, np.int32(0)), axis=1)


def kernel(backtrack_potentials, city_to_insert_probs, edge_to_insert_probs, current_tour):
    in_spec = pl.BlockSpec((BB, N), lambda i: (i, 0))
    out_spec = pl.BlockSpec((BB,), lambda i: (i,))
    out_shape = jax.ShapeDtypeStruct((B,), jnp.int32)
    bt, city, es, ee = pl.pallas_call(
        _body,
        grid=(B // BB,),
        in_specs=[in_spec] * 4,
        out_specs=[out_spec] * 4,
        out_shape=[out_shape] * 4,
    )(backtrack_potentials, city_to_insert_probs, edge_to_insert_probs, current_tour)
    return bt, city, jnp.stack([es, ee], axis=1)
